# Initial kernel scaffold; baseline (speedup 1.0000x reference)
import jax
import jax.numpy as jnp
from jax.experimental import pallas as pl


def kernel(xyz, features, W0, b0, W1, b1, W2, b2):
    raise NotImplementedError("write your pallas kernel here")


# speedup vs baseline: 17.3115x; 17.3115x over previous
import functools

import jax
import jax.numpy as jnp
from jax import lax
from jax.experimental import pallas as pl
from jax.experimental.pallas import tpu as pltpu
from jax.experimental.pallas import tpu_sc as plsc

B, N, NPOINT, NSAMPLE, CIN = 8, 4096, 1024, 32, 64
RADIUS = 0.2
R2 = RADIUS * RADIUS
ROWS = B * NPOINT * NSAMPLE
RELW = 16

NC, NS = 2, 16
NW = NC * NS
CPW = (B * NPOINT) // NW
CHUNK = 32
NCHUNK = CPW // CHUNK


def _fps_body(x_ref, y_ref, z_ref, nx_ref, ny_ref, nz_ref):
    x = x_ref[...]
    y = y_ref[...]
    z = z_ref[...]
    lanes = lax.broadcasted_iota(jnp.int32, (B, N), 1)
    out_lanes = lax.broadcasted_iota(jnp.int32, (B, NPOINT), 1)

    def body(i, carry):
        dists, far, nx, ny, nz = carry
        sel = lanes == far
        cx = jnp.sum(jnp.where(sel, x, 0.0), axis=1, keepdims=True)
        cy = jnp.sum(jnp.where(sel, y, 0.0), axis=1, keepdims=True)
        cz = jnp.sum(jnp.where(sel, z, 0.0), axis=1, keepdims=True)
        osel = out_lanes == i
        nx = jnp.where(osel, cx, nx)
        ny = jnp.where(osel, cy, ny)
        nz = jnp.where(osel, cz, nz)
        dx = x - cx
        dy = y - cy
        dz = z - cz
        d = (dx * dx + dy * dy) + dz * dz
        dists = jnp.minimum(dists, d)
        m = jnp.max(dists, axis=1, keepdims=True)
        far = jnp.min(jnp.where(dists == m, lanes, N), axis=1, keepdims=True)
        return dists, far, nx, ny, nz

    dists0 = jnp.full((B, N), 1e10, jnp.float32)
    far0 = jnp.zeros((B, 1), jnp.int32)
    z0 = jnp.zeros((B, NPOINT), jnp.float32)
    _, _, nx, ny, nz = lax.fori_loop(0, NPOINT, body, (dists0, far0, z0, z0, z0))
    nx_ref[...] = nx
    ny_ref[...] = ny
    nz_ref[...] = nz


def _fps(x, y, z):
    out = jax.ShapeDtypeStruct((B, NPOINT), jnp.float32)
    return pl.pallas_call(
        _fps_body,
        out_shape=(out, out, out),
    )(x, y, z)


def _ballquery_body(x_hbm, y_hbm, z_hbm, cx_hbm, cy_hbm, cz_hbm, feat_hbm,
                    featg_hbm, rel_hbm,
                    xv, yv, zv, cxv, cyv, czv, idxv, gidxv, relv, rowsv, sem):
    cid = lax.axis_index("c")
    sid = lax.axis_index("s")
    wid = sid * NC + cid
    b = wid // (NW // B)
    q = wid % (NW // B)

    pltpu.sync_copy(x_hbm.at[pl.ds(b * N, N)], xv)
    pltpu.sync_copy(y_hbm.at[pl.ds(b * N, N)], yv)
    pltpu.sync_copy(z_hbm.at[pl.ds(b * N, N)], zv)
    cbase = b * NPOINT + q * CPW
    pltpu.sync_copy(cx_hbm.at[pl.ds(cbase, CPW)], cxv)
    pltpu.sync_copy(cy_hbm.at[pl.ds(cbase, CPW)], cyv)
    pltpu.sync_copy(cz_hbm.at[pl.ds(cbase, CPW)], czv)

    zero16 = jnp.zeros((RELW,), jnp.float32)

    def zero_rel(j, _):
        relv[j, :] = zero16
        return 0

    lax.fori_loop(0, CHUNK * NSAMPLE, zero_rel, 0)

    iota16 = lax.iota(jnp.int32, 16)

    def splat_i(v):
        return jnp.broadcast_to(v, (16,)).astype(jnp.int32)

    def do_chunk(chunk, _):
        def do_cent(i, _):
            ci = chunk * CHUNK + i
            cxb = plsc.load_gather(cxv, [splat_i(ci)])
            cyb = plsc.load_gather(cyv, [splat_i(ci)])
            czb = plsc.load_gather(czv, [splat_i(ci)])
            slot0 = i * NSAMPLE

            def wcond(st):
                j, cnt = st
                return jnp.logical_and(j < N // 16, cnt < NSAMPLE)

            def wbody(st):
                j, cnt = st
                base = j * 16
                dx = xv[pl.ds(base, 16)] - cxb
                dy = yv[pl.ds(base, 16)] - cyb
                dz = zv[pl.ds(base, 16)] - czb
                d2 = (dx * dx + dy * dy) + dz * dz
                m = d2 < R2
                ones = m.astype(jnp.int32)
                pos = plsc.cumsum(ones) + cnt
                keep = jnp.logical_and(m, pos <= NSAMPLE)
                cols = iota16 + base
                plsc.store_scatter(idxv, [pos + (slot0 - 1)], cols, mask=keep)
                return j + 1, cnt + jnp.sum(ones)

            _, cnt = lax.while_loop(wcond, wbody, (jnp.int32(0), jnp.int32(0)))
            cntc = jnp.minimum(cnt, NSAMPLE)
            firstv = plsc.load_gather(idxv, [splat_i(slot0)])
            for h in range(NSAMPLE // 16):
                sl = pl.ds(slot0 + h * 16, 16)
                slots = iota16 + h * 16
                v = jnp.where(slots < cntc, idxv[sl], firstv)
                idxv[sl] = v
                gidxv[sl] = v + b * N
                gx = plsc.load_gather(xv, [v]) - cxb
                gy = plsc.load_gather(yv, [v]) - cyb
                gz = plsc.load_gather(zv, [v]) - czb
                rows16 = iota16 + (slot0 + h * 16)
                plsc.store_scatter(relv, [rows16, splat_i(0)], gx)
                plsc.store_scatter(relv, [rows16, splat_i(1)], gy)
                plsc.store_scatter(relv, [rows16, splat_i(2)], gz)
            return 0

        lax.fori_loop(0, CHUNK, do_cent, 0)
        rowb = wid * (CPW * NSAMPLE) + chunk * (CHUNK * NSAMPLE)
        pltpu.async_copy(feat_hbm.at[gidxv], rowsv, sem).wait()
        pltpu.sync_copy(rowsv, featg_hbm.at[pl.ds(rowb, CHUNK * NSAMPLE)])
        pltpu.sync_copy(relv, rel_hbm.at[pl.ds(rowb, CHUNK * NSAMPLE)])
        return 0

    lax.fori_loop(0, NCHUNK, do_chunk, 0)


def _ballquery(x, y, z, nx, ny, nz, feat_rows):
    mesh = plsc.VectorSubcoreMesh(core_axis_name="c", subcore_axis_name="s")
    fn = functools.partial(
        pl.kernel,
        mesh=mesh,
        compiler_params=pltpu.CompilerParams(
            needs_layout_passes=False, use_tc_tiling_on_sc=False),
        out_type=[
            jax.ShapeDtypeStruct((ROWS, CIN), jnp.float32),
            jax.ShapeDtypeStruct((ROWS, RELW), jnp.float32),
        ],
        scratch_types=[
            pltpu.VMEM((N,), jnp.float32),
            pltpu.VMEM((N,), jnp.float32),
            pltpu.VMEM((N,), jnp.float32),
            pltpu.VMEM((CPW,), jnp.float32),
            pltpu.VMEM((CPW,), jnp.float32),
            pltpu.VMEM((CPW,), jnp.float32),
            pltpu.VMEM((CHUNK * NSAMPLE,), jnp.int32),
            pltpu.VMEM((CHUNK * NSAMPLE,), jnp.int32),
            pltpu.VMEM((CHUNK * NSAMPLE, RELW), jnp.float32),
            pltpu.VMEM((CHUNK * NSAMPLE, CIN), jnp.float32),
            pltpu.SemaphoreType.DMA,
        ],
    )(_ballquery_body)
    return fn(
        x.reshape(B * N), y.reshape(B * N), z.reshape(B * N),
        nx.reshape(B * NPOINT), ny.reshape(B * NPOINT), nz.reshape(B * NPOINT),
        feat_rows,
    )


CROWS = 4096


def _mlp_body(feat_ref, rel_ref, w0f_ref, w0p_ref, b0_ref, w1_ref, b1_ref,
              w2_ref, b2_ref, out_ref):
    feat = feat_ref[...]
    rel = rel_ref[...]
    h = jnp.dot(feat, w0f_ref[...], preferred_element_type=jnp.float32)
    h += jnp.dot(rel, w0p_ref[...], preferred_element_type=jnp.float32)
    h = jnp.maximum(h + b0_ref[...], 0.0)
    h = jnp.maximum(
        jnp.dot(h, w1_ref[...], preferred_element_type=jnp.float32) + b1_ref[...], 0.0)
    h = jnp.maximum(
        jnp.dot(h, w2_ref[...], preferred_element_type=jnp.float32) + b2_ref[...], 0.0)
    h = h.reshape(CROWS // NSAMPLE, NSAMPLE, h.shape[-1])
    out_ref[...] = jnp.max(h, axis=1)


def _mlp(featg, rel, w0f, w0p, b0, w1, b1, w2, b2):
    cout = w2.shape[1]
    grid = (ROWS // CROWS,)
    wspec = lambda shp: pl.BlockSpec(shp, lambda i: (0,) * len(shp))
    return pl.pallas_call(
        _mlp_body,
        grid=grid,
        in_specs=[
            pl.BlockSpec((CROWS, CIN), lambda i: (i, 0)),
            pl.BlockSpec((CROWS, RELW), lambda i: (i, 0)),
            wspec((CIN, 64)),
            wspec((RELW, 64)),
            wspec((1, 64)),
            wspec((64, 64)),
            wspec((1, 64)),
            wspec((64, cout)),
            wspec((1, cout)),
        ],
        out_specs=pl.BlockSpec((CROWS // NSAMPLE, cout), lambda i: (i, 0)),
        out_shape=jax.ShapeDtypeStruct((ROWS // NSAMPLE, cout), jnp.float32),
    )(featg, rel, w0f, w0p, b0, w1, b1, w2, b2)


def kernel(xyz, features, W0, b0, W1, b1, W2, b2):
    x = xyz[:, :, 0]
    y = xyz[:, :, 1]
    z = xyz[:, :, 2]
    nx, ny, nz = _fps(x, y, z)
    new_xyz = jnp.stack([nx, ny, nz], axis=-1)

    feat_rows = jnp.transpose(features, (0, 2, 1)).reshape(B * N, CIN)
    featg, rel = _ballquery(x, y, z, nx, ny, nz, feat_rows)

    w0p = jnp.zeros((RELW, W0.shape[0]), jnp.float32).at[:3].set(W0[:, :3].T)
    pooled = _mlp(
        featg, rel,
        W0[:, 3:].T, w0p, b0[None, :],
        W1.T, b1[None, :],
        W2.T, b2[None, :],
    )
    new_features = jnp.transpose(pooled.reshape(B, NPOINT, -1), (0, 2, 1))
    return (new_xyz, new_features)


# speedup vs baseline: 35.2634x; 2.0370x over previous
import functools

import jax
import jax.numpy as jnp
from jax import lax
from jax.experimental import pallas as pl
from jax.experimental.pallas import tpu as pltpu
from jax.experimental.pallas import tpu_sc as plsc

B, N, NPOINT, NSAMPLE, CIN = 8, 4096, 1024, 32, 64
RADIUS = 0.2
R2 = RADIUS * RADIUS
ROWS = B * NPOINT * NSAMPLE
HW = 128

NC, NS = 2, 16
NW = NC * NS
HB = B // 2
WPB = NW // HB
CPW = NPOINT // WPB
CHUNK = 8
NCHUNK = CPW // CHUNK
CROWS_SC = CHUNK * NSAMPLE
HROWS = ROWS // 2


def _fps_body(x_ref, y_ref, z_ref, nx_ref, ny_ref, nz_ref):
    x = x_ref[...]
    y = y_ref[...]
    z = z_ref[...]
    lanes = lax.broadcasted_iota(jnp.int32, (B, N), 1)
    out_lanes = lax.broadcasted_iota(jnp.int32, (B, NPOINT), 1)

    def body(i, carry):
        dists, far, nx, ny, nz = carry
        sel = lanes == far
        cx = jnp.sum(jnp.where(sel, x, 0.0), axis=1, keepdims=True)
        cy = jnp.sum(jnp.where(sel, y, 0.0), axis=1, keepdims=True)
        cz = jnp.sum(jnp.where(sel, z, 0.0), axis=1, keepdims=True)
        osel = out_lanes == i
        nx = jnp.where(osel, cx, nx)
        ny = jnp.where(osel, cy, ny)
        nz = jnp.where(osel, cz, nz)
        dx = x - cx
        dy = y - cy
        dz = z - cz
        d = (dx * dx + dy * dy) + dz * dz
        dists = jnp.minimum(dists, d)
        m = jnp.max(dists, axis=1, keepdims=True)
        far = jnp.min(jnp.where(dists == m, lanes, N), axis=1, keepdims=True)
        return dists, far, nx, ny, nz

    dists0 = jnp.full((B, N), 1e10, jnp.float32)
    far0 = jnp.zeros((B, 1), jnp.int32)
    z0 = jnp.zeros((B, NPOINT), jnp.float32)
    _, _, nx, ny, nz = lax.fori_loop(0, NPOINT, body, (dists0, far0, z0, z0, z0))
    nx_ref[...] = nx
    ny_ref[...] = ny
    nz_ref[...] = nz


def _fps(x, y, z):
    out = jax.ShapeDtypeStruct((B, NPOINT), jnp.float32)
    return pl.pallas_call(
        _fps_body,
        out_shape=(out, out, out),
    )(x, y, z)


def _make_ballquery_body(bhalf):
  def _ballquery_body(x_hbm, y_hbm, z_hbm, cx_hbm, cy_hbm, cz_hbm, feat_hbm,
                    hcat_hbm,
                    xv, yv, zv, cxv, cyv, czv, idxv,
                    gidx0, gidx1, rows0, rows1,
                    gsem0, gsem1, wsem0, wsem1):
    cid = lax.axis_index("c")
    sid = lax.axis_index("s")
    wid = sid * NC + cid
    b = bhalf * HB + wid // WPB
    q = wid % WPB

    gidx = (gidx0, gidx1)
    rows = (rows0, rows1)
    gsem = (gsem0, gsem1)
    wsem = (wsem0, wsem1)

    pltpu.sync_copy(x_hbm.at[pl.ds(b * N, N)], xv)
    pltpu.sync_copy(y_hbm.at[pl.ds(b * N, N)], yv)
    pltpu.sync_copy(z_hbm.at[pl.ds(b * N, N)], zv)
    cbase = b * NPOINT + q * CPW
    pltpu.sync_copy(cx_hbm.at[pl.ds(cbase, CPW)], cxv)
    pltpu.sync_copy(cy_hbm.at[pl.ds(cbase, CPW)], cyv)
    pltpu.sync_copy(cz_hbm.at[pl.ds(cbase, CPW)], czv)

    iota16 = lax.iota(jnp.int32, 16)

    def splat_i(v):
        return jnp.broadcast_to(v, (16,)).astype(jnp.int32)

    GV = 16
    lane15 = jnp.full((16,), 15, jnp.int32)

    def select_chunk(c, bsel):
        def do_cent(i, _):
            ci = c * CHUNK + i
            cxb = plsc.load_gather(cxv, [splat_i(ci)])
            cyb = plsc.load_gather(cyv, [splat_i(ci)])
            czb = plsc.load_gather(czv, [splat_i(ci)])
            slot0 = i * NSAMPLE

            def ga_cond(st):
                g, notdone, _ = st
                return jnp.logical_and(g < N // (16 * GV), notdone)

            def ga_body(st):
                g, _, carry0 = st

                @plsc.parallel_loop(0, GV, carry=carry0)
                def carry(jj, carry):
                    base = (g * GV + jj) * 16
                    dx = xv[pl.ds(base, 16)] - cxb
                    dy = yv[pl.ds(base, 16)] - cyb
                    dz = zv[pl.ds(base, 16)] - czb
                    d2 = (dx * dx + dy * dy) + dz * dz
                    m = d2 < R2
                    pos = plsc.cumsum(m.astype(jnp.int32)) + carry
                    keep = jnp.logical_and(m, pos <= NSAMPLE)
                    cols = iota16 + base
                    plsc.store_scatter(idxv, [pos + (slot0 - 1)], cols,
                                       mask=keep)
                    return carry + plsc.all_reduce_population_count(m)

                return g + 1, jnp.logical_not(jnp.any(carry >= NSAMPLE)), carry

            _, _, carryf = lax.while_loop(
                ga_cond, ga_body,
                (jnp.int32(0), jnp.bool_(True), jnp.zeros((16,), jnp.int32)))
            cntc = jnp.minimum(jnp.max(carryf), NSAMPLE)
            firstv = plsc.load_gather(idxv, [splat_i(slot0)])
            for h in range(NSAMPLE // 16):
                sl = pl.ds(slot0 + h * 16, 16)
                slots = iota16 + h * 16
                v = jnp.where(slots < cntc, idxv[sl], firstv)
                idxv[sl] = v
                gidx[bsel][sl] = v + b * N
            return 0

        lax.fori_loop(0, CHUNK, do_cent, 0)

    def finish_chunk(c, ob):
        pltpu.make_async_copy(feat_hbm.at[gidx[ob]], rows[ob], gsem[ob]).wait()
        rowb = wid * (CPW * NSAMPLE) + c * CROWS_SC
        pltpu.async_copy(rows[ob], hcat_hbm.at[pl.ds(rowb, CROWS_SC)], wsem[ob])

    def wb_wait(c, bsel):
        rowb = wid * (CPW * NSAMPLE) + c * CROWS_SC
        pltpu.make_async_copy(
            rows[bsel], hcat_hbm.at[pl.ds(rowb, CROWS_SC)], wsem[bsel]).wait()

    def ring(c2, _):
        for bsel in range(2):
            ob = 1 - bsel
            c = c2 * 2 + bsel

            @pl.when(c2 >= 1)
            def _():
                wb_wait(c - 2, bsel)

            select_chunk(c, bsel)

            if bsel == 1:
                finish_chunk(c - 1, ob)
            else:

                @pl.when(c2 >= 1)
                def _():
                    finish_chunk(c - 1, ob)

            pltpu.async_copy(feat_hbm.at[gidx[bsel]], rows[bsel], gsem[bsel])
        return 0

    lax.fori_loop(0, NCHUNK // 2, ring, 0)
    finish_chunk(NCHUNK - 1, 1)
    wb_wait(NCHUNK - 2, 0)
    wb_wait(NCHUNK - 1, 1)

  return _ballquery_body


def _ballquery_half(bhalf, x, y, z, nx, ny, nz, feat128):
    mesh = plsc.VectorSubcoreMesh(core_axis_name="c", subcore_axis_name="s")
    fn = functools.partial(
        pl.kernel,
        mesh=mesh,
        compiler_params=pltpu.CompilerParams(needs_layout_passes=False),
        out_type=jax.ShapeDtypeStruct((HROWS, HW), jnp.float32),
        scratch_types=[
            pltpu.VMEM((N,), jnp.float32),
            pltpu.VMEM((N,), jnp.float32),
            pltpu.VMEM((N,), jnp.float32),
            pltpu.VMEM((CPW,), jnp.float32),
            pltpu.VMEM((CPW,), jnp.float32),
            pltpu.VMEM((CPW,), jnp.float32),
            pltpu.VMEM((CROWS_SC,), jnp.int32),
            pltpu.VMEM((CROWS_SC,), jnp.int32),
            pltpu.VMEM((CROWS_SC,), jnp.int32),
            pltpu.VMEM((CROWS_SC, HW), jnp.float32),
            pltpu.VMEM((CROWS_SC, HW), jnp.float32),
            pltpu.SemaphoreType.DMA,
            pltpu.SemaphoreType.DMA,
            pltpu.SemaphoreType.DMA,
            pltpu.SemaphoreType.DMA,
        ],
    )(_make_ballquery_body(bhalf))
    return fn(x, y, z, nx, ny, nz, feat128)


def _ballquery(x, y, z, nx, ny, nz, feat128):
    xf = x.reshape(B * N)
    yf = y.reshape(B * N)
    zf = z.reshape(B * N)
    nxf = nx.reshape(B * NPOINT)
    nyf = ny.reshape(B * NPOINT)
    nzf = nz.reshape(B * NPOINT)
    h0 = _ballquery_half(0, xf, yf, zf, nxf, nyf, nzf, feat128)
    h1 = _ballquery_half(1, xf, yf, zf, nxf, nyf, nzf, feat128)
    return h0, h1


CROWS = 4096


def _mlp_body(h_ref, cb_ref, w0_ref, w1_ref, b1_ref, w2_ref, b2_ref, out_ref):
    h = jnp.dot(h_ref[...], w0_ref[...], preferred_element_type=jnp.float32)
    h = h.reshape(CROWS // NSAMPLE, NSAMPLE, h.shape[-1]) + cb_ref[...][:, None, :]
    h = jnp.maximum(h, 0.0).reshape(CROWS, h.shape[-1])
    h = jnp.maximum(
        jnp.dot(h, w1_ref[...], preferred_element_type=jnp.float32) + b1_ref[...], 0.0)
    h = jnp.maximum(
        jnp.dot(h, w2_ref[...], preferred_element_type=jnp.float32) + b2_ref[...], 0.0)
    h = h.reshape(CROWS // NSAMPLE, NSAMPLE, h.shape[-1])
    out_ref[...] = jnp.max(h, axis=1)


def _mlp(hcat, cb, w0, w1, b1, w2, b2):
    cout = w2.shape[1]
    nrows = hcat.shape[0]
    grid = (nrows // CROWS,)
    wspec = lambda shp: pl.BlockSpec(shp, lambda i: (0,) * len(shp))
    return pl.pallas_call(
        _mlp_body,
        grid=grid,
        in_specs=[
            pl.BlockSpec((CROWS, HW), lambda i: (i, 0)),
            pl.BlockSpec((CROWS // NSAMPLE, 64), lambda i: (i, 0)),
            wspec((HW, 64)),
            wspec((64, 64)),
            wspec((1, 64)),
            wspec((64, cout)),
            wspec((1, cout)),
        ],
        out_specs=pl.BlockSpec((CROWS // NSAMPLE, cout), lambda i: (i, 0)),
        out_shape=jax.ShapeDtypeStruct((nrows // NSAMPLE, cout), jnp.float32),
    )(hcat, cb, w0, w1, b1, w2, b2)


def kernel(xyz, features, W0, b0, W1, b1, W2, b2):
    x = xyz[:, :, 0]
    y = xyz[:, :, 1]
    z = xyz[:, :, 2]
    nx, ny, nz = _fps(x, y, z)
    new_xyz = jnp.stack([nx, ny, nz], axis=-1)

    feat_t = jnp.transpose(features, (0, 2, 1)).reshape(B * N, CIN)
    feat128 = jnp.pad(
        jnp.concatenate([feat_t, xyz.reshape(B * N, 3)], axis=1),
        ((0, 0), (0, HW - CIN - 3)))
    h0, h1 = _ballquery(x, y, z, nx, ny, nz, feat128)

    w0cat = (jnp.zeros((HW, W0.shape[0]), jnp.float32)
             .at[:CIN].set(W0[:, 3:].T)
             .at[CIN:CIN + 3].set(W0[:, :3].T))
    cb = b0[None, :] - new_xyz.reshape(B * NPOINT, 3) @ W0[:, :3].T
    args = (w0cat, W1.T, b1[None, :], W2.T, b2[None, :])
    half = (B * NPOINT) // 2
    pooled = jnp.concatenate(
        [_mlp(h0, cb[:half], *args), _mlp(h1, cb[half:], *args)], axis=0)
    new_features = jnp.transpose(pooled.reshape(B, NPOINT, -1), (0, 2, 1))
    return (new_xyz, new_features)


# speedup vs baseline: 35.3129x; 1.0014x over previous
import functools

import jax
import jax.numpy as jnp
from jax import lax
from jax.experimental import pallas as pl
from jax.experimental.pallas import tpu as pltpu
from jax.experimental.pallas import tpu_sc as plsc

B, N, NPOINT, NSAMPLE, CIN = 8, 4096, 1024, 32, 64
RADIUS = 0.2
R2 = RADIUS * RADIUS
ROWS = B * NPOINT * NSAMPLE
HW = 128

NC, NS = 2, 16
NW = NC * NS
HB = B // 2
WPB = NW // HB
CPW = NPOINT // WPB
CHUNK = 8
NCHUNK = CPW // CHUNK
CROWS_SC = CHUNK * NSAMPLE
HROWS = ROWS // 2


def _fps_body(x_ref, y_ref, z_ref, nx_ref, ny_ref, nz_ref):
    x = x_ref[...]
    y = y_ref[...]
    z = z_ref[...]
    lanes = lax.broadcasted_iota(jnp.int32, (B, N), 1)
    out_lanes = lax.broadcasted_iota(jnp.int32, (B, NPOINT), 1)

    def body(i, carry):
        dists, far, nx, ny, nz = carry
        sel = lanes == far
        cx = jnp.sum(jnp.where(sel, x, 0.0), axis=1, keepdims=True)
        cy = jnp.sum(jnp.where(sel, y, 0.0), axis=1, keepdims=True)
        cz = jnp.sum(jnp.where(sel, z, 0.0), axis=1, keepdims=True)
        osel = out_lanes == i
        nx = jnp.where(osel, cx, nx)
        ny = jnp.where(osel, cy, ny)
        nz = jnp.where(osel, cz, nz)
        dx = x - cx
        dy = y - cy
        dz = z - cz
        d = (dx * dx + dz * dz) + dy * dy
        dists = jnp.minimum(dists, d)
        m = jnp.max(dists, axis=1, keepdims=True)
        far = jnp.min(jnp.where(dists == m, lanes, N), axis=1, keepdims=True)
        return dists, far, nx, ny, nz

    dists0 = jnp.full((B, N), 1e10, jnp.float32)
    far0 = jnp.zeros((B, 1), jnp.int32)
    z0 = jnp.zeros((B, NPOINT), jnp.float32)
    _, _, nx, ny, nz = lax.fori_loop(0, NPOINT, body, (dists0, far0, z0, z0, z0))
    nx_ref[...] = nx
    ny_ref[...] = ny
    nz_ref[...] = nz


def _fps(x, y, z):
    out = jax.ShapeDtypeStruct((B, NPOINT), jnp.float32)
    return pl.pallas_call(
        _fps_body,
        out_shape=(out, out, out),
    )(x, y, z)


def _make_ballquery_body(bhalf):
  def _ballquery_body(x_hbm, y_hbm, z_hbm, cx_hbm, cy_hbm, cz_hbm, feat_hbm,
                    hcat_hbm,
                    xv, yv, zv, cxv, cyv, czv, idxv,
                    gidx0, gidx1, rows0, rows1,
                    gsem0, gsem1, wsem0, wsem1):
    cid = lax.axis_index("c")
    sid = lax.axis_index("s")
    wid = sid * NC + cid
    b = bhalf * HB + wid // WPB
    q = wid % WPB

    gidx = (gidx0, gidx1)
    rows = (rows0, rows1)
    gsem = (gsem0, gsem1)
    wsem = (wsem0, wsem1)

    pltpu.sync_copy(x_hbm.at[pl.ds(b * N, N)], xv)
    pltpu.sync_copy(y_hbm.at[pl.ds(b * N, N)], yv)
    pltpu.sync_copy(z_hbm.at[pl.ds(b * N, N)], zv)
    cbase = b * NPOINT + q * CPW
    pltpu.sync_copy(cx_hbm.at[pl.ds(cbase, CPW)], cxv)
    pltpu.sync_copy(cy_hbm.at[pl.ds(cbase, CPW)], cyv)
    pltpu.sync_copy(cz_hbm.at[pl.ds(cbase, CPW)], czv)

    iota16 = lax.iota(jnp.int32, 16)

    def splat_i(v):
        return jnp.broadcast_to(v, (16,)).astype(jnp.int32)

    GV = 16
    lane15 = jnp.full((16,), 15, jnp.int32)

    def select_chunk(c, bsel):
        def do_cent(i, _):
            ci = c * CHUNK + i
            cxb = plsc.load_gather(cxv, [splat_i(ci)])
            cyb = plsc.load_gather(cyv, [splat_i(ci)])
            czb = plsc.load_gather(czv, [splat_i(ci)])
            slot0 = i * NSAMPLE

            def ga_cond(st):
                g, notdone, _ = st
                return jnp.logical_and(g < N // (16 * GV), notdone)

            def ga_body(st):
                g, _, carry0 = st

                @plsc.parallel_loop(0, GV, carry=carry0)
                def carry(jj, carry):
                    base = (g * GV + jj) * 16
                    dx = xv[pl.ds(base, 16)] - cxb
                    dy = yv[pl.ds(base, 16)] - cyb
                    dz = zv[pl.ds(base, 16)] - czb
                    d2 = (dx * dx + dz * dz) + dy * dy
                    m = d2 < R2
                    pos = plsc.cumsum(m.astype(jnp.int32)) + carry
                    keep = jnp.logical_and(m, pos <= NSAMPLE)
                    cols = iota16 + base
                    plsc.store_scatter(idxv, [pos + (slot0 - 1)], cols,
                                       mask=keep)
                    return carry + plsc.all_reduce_population_count(m)

                return g + 1, jnp.logical_not(jnp.any(carry >= NSAMPLE)), carry

            _, _, carryf = lax.while_loop(
                ga_cond, ga_body,
                (jnp.int32(0), jnp.bool_(True), jnp.zeros((16,), jnp.int32)))
            cntc = jnp.minimum(jnp.max(carryf), NSAMPLE)
            firstv = plsc.load_gather(idxv, [splat_i(slot0)])
            for h in range(NSAMPLE // 16):
                sl = pl.ds(slot0 + h * 16, 16)
                slots = iota16 + h * 16
                v = jnp.where(slots < cntc, idxv[sl], firstv)
                idxv[sl] = v
                gidx[bsel][sl] = v + b * N
            return 0

        lax.fori_loop(0, CHUNK, do_cent, 0)

    def finish_chunk(c, ob):
        pltpu.make_async_copy(feat_hbm.at[gidx[ob]], rows[ob], gsem[ob]).wait()
        rowb = wid * (CPW * NSAMPLE) + c * CROWS_SC
        pltpu.async_copy(rows[ob], hcat_hbm.at[pl.ds(rowb, CROWS_SC)], wsem[ob])

    def wb_wait(c, bsel):
        rowb = wid * (CPW * NSAMPLE) + c * CROWS_SC
        pltpu.make_async_copy(
            rows[bsel], hcat_hbm.at[pl.ds(rowb, CROWS_SC)], wsem[bsel]).wait()

    def ring(c2, _):
        for bsel in range(2):
            ob = 1 - bsel
            c = c2 * 2 + bsel

            @pl.when(c2 >= 1)
            def _():
                wb_wait(c - 2, bsel)

            select_chunk(c, bsel)

            if bsel == 1:
                finish_chunk(c - 1, ob)
            else:

                @pl.when(c2 >= 1)
                def _():
                    finish_chunk(c - 1, ob)

            pltpu.async_copy(feat_hbm.at[gidx[bsel]], rows[bsel], gsem[bsel])
        return 0

    lax.fori_loop(0, NCHUNK // 2, ring, 0)
    finish_chunk(NCHUNK - 1, 1)
    wb_wait(NCHUNK - 2, 0)
    wb_wait(NCHUNK - 1, 1)

  return _ballquery_body


def _ballquery_half(bhalf, x, y, z, nx, ny, nz, feat128):
    mesh = plsc.VectorSubcoreMesh(core_axis_name="c", subcore_axis_name="s")
    fn = functools.partial(
        pl.kernel,
        mesh=mesh,
        compiler_params=pltpu.CompilerParams(needs_layout_passes=False),
        out_type=jax.ShapeDtypeStruct((HROWS, HW), jnp.float32),
        scratch_types=[
            pltpu.VMEM((N,), jnp.float32),
            pltpu.VMEM((N,), jnp.float32),
            pltpu.VMEM((N,), jnp.float32),
            pltpu.VMEM((CPW,), jnp.float32),
            pltpu.VMEM((CPW,), jnp.float32),
            pltpu.VMEM((CPW,), jnp.float32),
            pltpu.VMEM((CROWS_SC,), jnp.int32),
            pltpu.VMEM((CROWS_SC,), jnp.int32),
            pltpu.VMEM((CROWS_SC,), jnp.int32),
            pltpu.VMEM((CROWS_SC, HW), jnp.float32),
            pltpu.VMEM((CROWS_SC, HW), jnp.float32),
            pltpu.SemaphoreType.DMA,
            pltpu.SemaphoreType.DMA,
            pltpu.SemaphoreType.DMA,
            pltpu.SemaphoreType.DMA,
        ],
    )(_make_ballquery_body(bhalf))
    return fn(x, y, z, nx, ny, nz, feat128)


def _ballquery(x, y, z, nx, ny, nz, feat128):
    xf = x.reshape(B * N)
    yf = y.reshape(B * N)
    zf = z.reshape(B * N)
    nxf = nx.reshape(B * NPOINT)
    nyf = ny.reshape(B * NPOINT)
    nzf = nz.reshape(B * NPOINT)
    h0 = _ballquery_half(0, xf, yf, zf, nxf, nyf, nzf, feat128)
    h1 = _ballquery_half(1, xf, yf, zf, nxf, nyf, nzf, feat128)
    return h0, h1


CROWS = 4096


def _mlp_body(h_ref, cb_ref, w0_ref, w1_ref, b1_ref, w2_ref, b2_ref, out_ref):
    h = jnp.dot(h_ref[...], w0_ref[...], preferred_element_type=jnp.float32)
    h = h.reshape(CROWS // NSAMPLE, NSAMPLE, h.shape[-1]) + cb_ref[...][:, None, :]
    h = jnp.maximum(h, 0.0).reshape(CROWS, h.shape[-1])
    h = jnp.maximum(
        jnp.dot(h, w1_ref[...], preferred_element_type=jnp.float32) + b1_ref[...], 0.0)
    h = jnp.maximum(
        jnp.dot(h, w2_ref[...], preferred_element_type=jnp.float32) + b2_ref[...], 0.0)
    h = h.reshape(CROWS // NSAMPLE, NSAMPLE, h.shape[-1])
    out_ref[...] = jnp.max(h, axis=1)


def _mlp(hcat, cb, w0, w1, b1, w2, b2):
    cout = w2.shape[1]
    nrows = hcat.shape[0]
    grid = (nrows // CROWS,)
    wspec = lambda shp: pl.BlockSpec(shp, lambda i: (0,) * len(shp))
    return pl.pallas_call(
        _mlp_body,
        grid=grid,
        in_specs=[
            pl.BlockSpec((CROWS, HW), lambda i: (i, 0)),
            pl.BlockSpec((CROWS // NSAMPLE, 64), lambda i: (i, 0)),
            wspec((HW, 64)),
            wspec((64, 64)),
            wspec((1, 64)),
            wspec((64, cout)),
            wspec((1, cout)),
        ],
        out_specs=pl.BlockSpec((CROWS // NSAMPLE, cout), lambda i: (i, 0)),
        out_shape=jax.ShapeDtypeStruct((nrows // NSAMPLE, cout), jnp.float32),
    )(hcat, cb, w0, w1, b1, w2, b2)


def kernel(xyz, features, W0, b0, W1, b1, W2, b2):
    x = xyz[:, :, 0]
    y = xyz[:, :, 1]
    z = xyz[:, :, 2]
    nx, ny, nz = _fps(x, y, z)
    new_xyz = jnp.stack([nx, ny, nz], axis=-1)

    feat_t = jnp.transpose(features, (0, 2, 1)).reshape(B * N, CIN)
    feat128 = jnp.pad(
        jnp.concatenate([feat_t, xyz.reshape(B * N, 3)], axis=1),
        ((0, 0), (0, HW - CIN - 3)))
    h0, h1 = _ballquery(x, y, z, nx, ny, nz, feat128)

    w0cat = (jnp.zeros((HW, W0.shape[0]), jnp.float32)
             .at[:CIN].set(W0[:, 3:].T)
             .at[CIN:CIN + 3].set(W0[:, :3].T))
    cb = b0[None, :] - new_xyz.reshape(B * NPOINT, 3) @ W0[:, :3].T
    args = (w0cat, W1.T, b1[None, :], W2.T, b2[None, :])
    half = (B * NPOINT) // 2
    pooled = jnp.concatenate(
        [_mlp(h0, cb[:half], *args), _mlp(h1, cb[half:], *args)], axis=0)
    new_features = jnp.transpose(pooled.reshape(B, NPOINT, -1), (0, 2, 1))
    return (new_xyz, new_features)
